# Initial kernel scaffold; baseline (speedup 1.0000x reference)
import functools

import jax
import jax.numpy as jnp
from jax import lax
from jax.experimental import pallas as pl
from jax.experimental.pallas import tpu as pltpu
from jax.experimental.pallas import tpu_sc as plsc

N = 10000
D = 128
N_GRAPHS = 64
N_CLASSES = 16

NC = 2
NS = 16
NW = NC * NS
CH = 128
CHUNKS = 80
E_PAD = NW * CHUNKS * CH
ROWS_PER_TILE = N // NS


def _dot_t(a, b):
  return lax.dot_general(a, b, (((1,), (1,)), ((), ())),
                         preferred_element_type=jnp.float32,
                         precision=lax.Precision.HIGHEST)


def _edge_body(src_hbm, dst_hbm, ew_hbm, h_hbm, out0_hbm, out1_hbm,
               src_v, dst_v, ew_v, rows_v, acc_sh, sem):
  cid = lax.axis_index("c")
  sid = lax.axis_index("s")
  wid = sid * NC + cid

  def _zrow(e, _):
    for k in range(8):
      rows_v[e, pl.ds(k * 16, 16)] = jnp.zeros((16,), jnp.float32)
    return 0
  lax.fori_loop(0, 125, _zrow, 0, unroll=4)
  for i in range(5):
    pltpu.sync_copy(rows_v.at[pl.ds(0, 125)],
                    acc_sh.at[pl.ds(sid * ROWS_PER_TILE + i * 125, 125)])

  pltpu.sync_copy(src_hbm.at[pl.ds(wid * CHUNKS, CHUNKS)], src_v)
  pltpu.sync_copy(dst_hbm.at[pl.ds(wid * CHUNKS, CHUNKS)], dst_v)
  pltpu.sync_copy(ew_hbm.at[pl.ds(wid * CHUNKS, CHUNKS)], ew_v)

  plsc.subcore_barrier()

  def _chunk(i, _):
    pltpu.async_copy(h_hbm.at[src_v.at[i]], rows_v, sem).wait()

    def _scale(e, _):
      w = ew_v[i, e]
      for k in range(8):
        rows_v[e, pl.ds(k * 16, 16)] = rows_v[e, pl.ds(k * 16, 16)] * w
      return 0
    lax.fori_loop(0, CH, _scale, 0, unroll=4)

    pltpu.sync_copy(rows_v, acc_sh.at[dst_v.at[i]], add=True)
    return 0

  lax.fori_loop(0, CHUNKS, _chunk, 0)
  plsc.subcore_barrier()

  @pl.when(cid == 0)
  def _():
    pltpu.sync_copy(acc_sh.at[pl.ds(sid * ROWS_PER_TILE, ROWS_PER_TILE)],
                    out0_hbm.at[pl.ds(sid * ROWS_PER_TILE, ROWS_PER_TILE)])

  @pl.when(cid == 1)
  def _():
    pltpu.sync_copy(acc_sh.at[pl.ds(sid * ROWS_PER_TILE, ROWS_PER_TILE)],
                    out1_hbm.at[pl.ds(sid * ROWS_PER_TILE, ROWS_PER_TILE)])


_edge_aggregate = functools.partial(
    pl.kernel,
    out_type=(jax.ShapeDtypeStruct((N, D), jnp.float32),
              jax.ShapeDtypeStruct((N, D), jnp.float32)),
    mesh=plsc.VectorSubcoreMesh(core_axis_name="c", subcore_axis_name="s"),
    scratch_types=[
        pltpu.VMEM((CHUNKS, CH), jnp.int32),
        pltpu.VMEM((CHUNKS, CH), jnp.int32),
        pltpu.VMEM((CHUNKS, CH), jnp.float32),
        pltpu.VMEM((CH, D), jnp.float32),
        pltpu.VMEM_SHARED((N, D), jnp.float32),
        pltpu.SemaphoreType.DMA,
    ],
)(_edge_body)


_MB = 1000
_GRID = N // _MB


def _proj_body(x_ref, w_ref, b_ref, o_ref):
  o_ref[...] = _dot_t(x_ref[...], w_ref[...]) + b_ref[...]


def _proj(x, w, b2):
  return pl.pallas_call(
      _proj_body,
      grid=(_GRID,),
      in_specs=[
          pl.BlockSpec((_MB, D), lambda i: (i, 0)),
          pl.BlockSpec((D, D), lambda i: (0, 0)),
          pl.BlockSpec((1, D), lambda i: (0, 0)),
      ],
      out_specs=pl.BlockSpec((_MB, D), lambda i: (i, 0)),
      out_shape=jax.ShapeDtypeStruct((N, D), jnp.float32),
  )(x, w, b2)


def _combine_body(p0_ref, p1_ref, h_ref, wrel_ref, brel_ref, wroot_ref, o_ref):
  aggr = p0_ref[...] + p1_ref[...]
  t = _dot_t(aggr, wrel_ref[...]) + _dot_t(h_ref[...], wroot_ref[...]) \
      + brel_ref[...]
  o_ref[...] = jnp.maximum(t, 0.0)


def _combine(p0, p1, h, wrel, brel2, wroot):
  return pl.pallas_call(
      _combine_body,
      grid=(_GRID,),
      in_specs=[
          pl.BlockSpec((_MB, D), lambda i: (i, 0)),
          pl.BlockSpec((_MB, D), lambda i: (i, 0)),
          pl.BlockSpec((_MB, D), lambda i: (i, 0)),
          pl.BlockSpec((D, D), lambda i: (0, 0)),
          pl.BlockSpec((1, D), lambda i: (0, 0)),
          pl.BlockSpec((D, D), lambda i: (0, 0)),
      ],
      out_specs=pl.BlockSpec((_MB, D), lambda i: (i, 0)),
      out_shape=jax.ShapeDtypeStruct((N, D), jnp.float32),
  )(p0, p1, h, wrel, brel2, wroot)


def _final_body(h_ref, b2_ref, wcls_ref, bcls_ref, logits_ref, g_ref,
                sums_acc, cnt_acc):
  i = pl.program_id(0)

  @pl.when(i == 0)
  def _():
    sums_acc[...] = jnp.zeros_like(sums_acc)
    cnt_acc[...] = jnp.zeros_like(cnt_acc)

  onehot = (b2_ref[...] == lax.broadcasted_iota(jnp.int32, (1, N_GRAPHS), 1)
            ).astype(jnp.float32)
  sums_acc[...] += lax.dot_general(
      onehot, h_ref[...], (((0,), (0,)), ((), ())),
      preferred_element_type=jnp.float32, precision=lax.Precision.HIGHEST)
  cnt_acc[...] += lax.dot_general(
      onehot, jnp.ones((_MB, 1), jnp.float32), (((0,), (0,)), ((), ())),
      preferred_element_type=jnp.float32, precision=lax.Precision.HIGHEST)

  @pl.when(i == _GRID - 1)
  def _():
    g = sums_acc[...] / jnp.maximum(cnt_acc[...], 1.0)
    g_ref[...] = g
    logits_ref[...] = _dot_t(g, wcls_ref[...]) + bcls_ref[...]


def _final(h, batch2d, wcls, bcls2):
  return pl.pallas_call(
      _final_body,
      grid=(_GRID,),
      in_specs=[
          pl.BlockSpec((_MB, D), lambda i: (i, 0)),
          pl.BlockSpec((_MB, 1), lambda i: (i, 0)),
          pl.BlockSpec((N_CLASSES, D), lambda i: (0, 0)),
          pl.BlockSpec((1, N_CLASSES), lambda i: (0, 0)),
      ],
      out_specs=[
          pl.BlockSpec((N_GRAPHS, N_CLASSES), lambda i: (0, 0)),
          pl.BlockSpec((N_GRAPHS, D), lambda i: (0, 0)),
      ],
      out_shape=[
          jax.ShapeDtypeStruct((N_GRAPHS, N_CLASSES), jnp.float32),
          jax.ShapeDtypeStruct((N_GRAPHS, D), jnp.float32),
      ],
      scratch_shapes=[
          pltpu.VMEM((N_GRAPHS, D), jnp.float32),
          pltpu.VMEM((N_GRAPHS, 1), jnp.float32),
      ],
  )(h, batch2d, wcls, bcls2)


def kernel(x_nodes, edge_index, edge_weight, batch, W_proj, b_proj,
           Wrel0, brel0, Wroot0, Wrel1, brel1, Wroot1, Wcls, bcls):
  pad = E_PAD - edge_weight.shape[0]
  src = jnp.concatenate([edge_index[0], jnp.zeros((pad,), jnp.int32)])
  dst = jnp.concatenate([edge_index[1], jnp.zeros((pad,), jnp.int32)])
  ew = jnp.concatenate([edge_weight, jnp.zeros((pad,), jnp.float32)])
  src2 = src.reshape(NW * CHUNKS, CH)
  dst2 = dst.reshape(NW * CHUNKS, CH)
  ew2 = ew.reshape(NW * CHUNKS, CH)
  batch2d = batch[:, None]

  h0 = _proj(x_nodes, W_proj, b_proj[None, :])
  p0a, p0b = _edge_aggregate(src2, dst2, ew2, h0)
  h1 = _combine(p0a, p0b, h0, Wrel0, brel0[None, :], Wroot0)
  p1a, p1b = _edge_aggregate(src2, dst2, ew2, h1)
  h2 = _combine(p1a, p1b, h1, Wrel1, brel1[None, :], Wroot1)
  logits, g = _final(h2, batch2d, Wcls, bcls[None, :])
  return logits, g


# speedup vs baseline: 2.8321x; 2.8321x over previous
import functools

import jax
import jax.numpy as jnp
from jax import lax
from jax.experimental import pallas as pl
from jax.experimental.pallas import tpu as pltpu
from jax.experimental.pallas import tpu_sc as plsc

N = 10000
N_PAD = 10240
D = 128
N_GRAPHS = 64
N_CLASSES = 16

NC = 2
NS = 16
NW = NC * NS
CH = 128
CHUNKS = 80
E_PAD = NW * CHUNKS * CH
ROWS_PER_TILE = N_PAD // NS


def _dot_t(a, b):
  return lax.dot_general(a, b, (((1,), (1,)), ((), ())),
                         preferred_element_type=jnp.float32,
                         precision=lax.Precision.HIGHEST)


def _edge_body(src_hbm, dst_hbm, ew_hbm, h_hbm, out0_hbm, out1_hbm,
               src_v, dst_v, ew_v, rows_v, acc_sh, sem):
  cid = lax.axis_index("c")
  sid = lax.axis_index("s")
  wid = sid * NC + cid

  def _zrow(e, _):
    for k in range(8):
      rows_v[e, pl.ds(k * 16, 16)] = jnp.zeros((16,), jnp.float32)
    return 0
  lax.fori_loop(0, CH, _zrow, 0, unroll=4)
  for i in range(ROWS_PER_TILE // CH):
    pltpu.sync_copy(rows_v,
                    acc_sh.at[pl.ds(sid * ROWS_PER_TILE + i * CH, CH)])

  pltpu.sync_copy(src_hbm.at[pl.ds(wid * CHUNKS, CHUNKS)], src_v)
  pltpu.sync_copy(dst_hbm.at[pl.ds(wid * CHUNKS, CHUNKS)], dst_v)
  pltpu.sync_copy(ew_hbm.at[pl.ds(wid * CHUNKS, CHUNKS)], ew_v)

  plsc.subcore_barrier()

  def _chunk(i, _):
    pltpu.async_copy(h_hbm.at[src_v.at[i]], rows_v, sem).wait()

    def _scale(j, _):
      wv = ew_v[i, pl.ds(j * 16, 16)]
      for t in range(16):
        w = wv[t]
        e = j * 16 + t
        for k in range(8):
          rows_v[e, pl.ds(k * 16, 16)] = rows_v[e, pl.ds(k * 16, 16)] * w
      return 0
    lax.fori_loop(0, CH // 16, _scale, 0)

    pltpu.sync_copy(rows_v, acc_sh.at[dst_v.at[i]], add=True)
    return 0

  lax.fori_loop(0, CHUNKS, _chunk, 0)
  plsc.subcore_barrier()

  @pl.when(cid == 0)
  def _():
    pltpu.sync_copy(acc_sh.at[pl.ds(sid * ROWS_PER_TILE, ROWS_PER_TILE)],
                    out0_hbm.at[pl.ds(sid * ROWS_PER_TILE, ROWS_PER_TILE)])

  @pl.when(cid == 1)
  def _():
    pltpu.sync_copy(acc_sh.at[pl.ds(sid * ROWS_PER_TILE, ROWS_PER_TILE)],
                    out1_hbm.at[pl.ds(sid * ROWS_PER_TILE, ROWS_PER_TILE)])


_edge_aggregate = functools.partial(
    pl.kernel,
    out_type=(jax.ShapeDtypeStruct((N_PAD, D), jnp.float32),
              jax.ShapeDtypeStruct((N_PAD, D), jnp.float32)),
    mesh=plsc.VectorSubcoreMesh(core_axis_name="c", subcore_axis_name="s"),
    scratch_types=[
        pltpu.VMEM((CHUNKS, CH), jnp.int32),
        pltpu.VMEM((CHUNKS, CH), jnp.int32),
        pltpu.VMEM((CHUNKS, CH), jnp.float32),
        pltpu.VMEM((CH, D), jnp.float32),
        pltpu.VMEM_SHARED((N_PAD, D), jnp.float32),
        pltpu.SemaphoreType.DMA,
    ],
)(_edge_body)


_MB = 1024
_GRID = N_PAD // _MB


def _proj_body(x_ref, w_ref, b_ref, o_ref):
  o_ref[...] = _dot_t(x_ref[...], w_ref[...]) + b_ref[...]


def _proj(x, w, b2):
  return pl.pallas_call(
      _proj_body,
      grid=(_GRID,),
      in_specs=[
          pl.BlockSpec((_MB, D), lambda i: (i, 0)),
          pl.BlockSpec((D, D), lambda i: (0, 0)),
          pl.BlockSpec((1, D), lambda i: (0, 0)),
      ],
      out_specs=pl.BlockSpec((_MB, D), lambda i: (i, 0)),
      out_shape=jax.ShapeDtypeStruct((N_PAD, D), jnp.float32),
  )(x, w, b2)


def _combine_body(p0_ref, p1_ref, h_ref, wrel_ref, brel_ref, wroot_ref, o_ref):
  aggr = p0_ref[...] + p1_ref[...]
  t = _dot_t(aggr, wrel_ref[...]) + _dot_t(h_ref[...], wroot_ref[...]) \
      + brel_ref[...]
  o_ref[...] = jnp.maximum(t, 0.0)


def _combine(p0, p1, h, wrel, brel2, wroot):
  return pl.pallas_call(
      _combine_body,
      grid=(_GRID,),
      in_specs=[
          pl.BlockSpec((_MB, D), lambda i: (i, 0)),
          pl.BlockSpec((_MB, D), lambda i: (i, 0)),
          pl.BlockSpec((_MB, D), lambda i: (i, 0)),
          pl.BlockSpec((D, D), lambda i: (0, 0)),
          pl.BlockSpec((1, D), lambda i: (0, 0)),
          pl.BlockSpec((D, D), lambda i: (0, 0)),
      ],
      out_specs=pl.BlockSpec((_MB, D), lambda i: (i, 0)),
      out_shape=jax.ShapeDtypeStruct((N_PAD, D), jnp.float32),
  )(p0, p1, h, wrel, brel2, wroot)


def _final_body(h_ref, b2_ref, wcls_ref, bcls_ref, logits_ref, g_ref,
                sums_acc, cnt_acc):
  i = pl.program_id(0)

  @pl.when(i == 0)
  def _():
    sums_acc[...] = jnp.zeros_like(sums_acc)
    cnt_acc[...] = jnp.zeros_like(cnt_acc)

  onehot = (b2_ref[...] == lax.broadcasted_iota(jnp.int32, (1, N_GRAPHS), 1)
            ).astype(jnp.float32)
  sums_acc[...] += lax.dot_general(
      onehot, h_ref[...], (((0,), (0,)), ((), ())),
      preferred_element_type=jnp.float32, precision=lax.Precision.HIGHEST)
  cnt_acc[...] += lax.dot_general(
      onehot, jnp.ones((_MB, 1), jnp.float32), (((0,), (0,)), ((), ())),
      preferred_element_type=jnp.float32, precision=lax.Precision.HIGHEST)

  @pl.when(i == _GRID - 1)
  def _():
    g = sums_acc[...] / jnp.maximum(cnt_acc[...], 1.0)
    g_ref[...] = g
    logits_ref[...] = _dot_t(g, wcls_ref[...]) + bcls_ref[...]


def _final(h, batch2d, wcls, bcls2):
  return pl.pallas_call(
      _final_body,
      grid=(_GRID,),
      in_specs=[
          pl.BlockSpec((_MB, D), lambda i: (i, 0)),
          pl.BlockSpec((_MB, 1), lambda i: (i, 0)),
          pl.BlockSpec((N_CLASSES, D), lambda i: (0, 0)),
          pl.BlockSpec((1, N_CLASSES), lambda i: (0, 0)),
      ],
      out_specs=[
          pl.BlockSpec((N_GRAPHS, N_CLASSES), lambda i: (0, 0)),
          pl.BlockSpec((N_GRAPHS, D), lambda i: (0, 0)),
      ],
      out_shape=[
          jax.ShapeDtypeStruct((N_GRAPHS, N_CLASSES), jnp.float32),
          jax.ShapeDtypeStruct((N_GRAPHS, D), jnp.float32),
      ],
      scratch_shapes=[
          pltpu.VMEM((N_GRAPHS, D), jnp.float32),
          pltpu.VMEM((N_GRAPHS, 1), jnp.float32),
      ],
  )(h, batch2d, wcls, bcls2)


def kernel(x_nodes, edge_index, edge_weight, batch, W_proj, b_proj,
           Wrel0, brel0, Wroot0, Wrel1, brel1, Wroot1, Wcls, bcls):
  pad = E_PAD - edge_weight.shape[0]
  src = jnp.concatenate([edge_index[0], jnp.zeros((pad,), jnp.int32)])
  dst = jnp.concatenate([edge_index[1], jnp.zeros((pad,), jnp.int32)])
  ew = jnp.concatenate([edge_weight, jnp.zeros((pad,), jnp.float32)])
  src2 = src.reshape(NW * CHUNKS, CH)
  dst2 = dst.reshape(NW * CHUNKS, CH)
  ew2 = ew.reshape(NW * CHUNKS, CH)
  x_nodes = jnp.pad(x_nodes, ((0, N_PAD - N), (0, 0)))
  batch2d = jnp.pad(batch, (0, N_PAD - N), constant_values=N_GRAPHS)[:, None]

  h0 = _proj(x_nodes, W_proj, b_proj[None, :])
  p0a, p0b = _edge_aggregate(src2, dst2, ew2, h0)
  h1 = _combine(p0a, p0b, h0, Wrel0, brel0[None, :], Wroot0)
  p1a, p1b = _edge_aggregate(src2, dst2, ew2, h1)
  h2 = _combine(p1a, p1b, h1, Wrel1, brel1[None, :], Wroot1)
  logits, g = _final(h2, batch2d, Wcls, bcls[None, :])
  return logits, g


# speedup vs baseline: 6.6393x; 2.3443x over previous
import functools

import jax
import jax.numpy as jnp
from jax import lax
from jax.experimental import pallas as pl
from jax.experimental.pallas import tpu as pltpu
from jax.experimental.pallas import tpu_sc as plsc

N = 10000
N_PAD = 10240
D = 128
N_GRAPHS = 64
N_CLASSES = 16

NC = 2
NS = 16
NW = NC * NS
CH = 112
CHUNKS = 90
GRP = 3
E_PAD = NW * CHUNKS * CH
ROWS_PER_TILE = N_PAD // NS


def _dot_t(a, b):
  return lax.dot_general(a, b, (((1,), (1,)), ((), ())),
                         preferred_element_type=jnp.float32,
                         precision=lax.Precision.HIGHEST)


def _edge_body(comb_hbm, ew_hbm, h_hbm, out0_hbm, out1_hbm,
               cb0, cb1, cb2, eb0, eb1, eb2, rows0, rows1, acc_sh,
               lc0, lc1, lc2, le0, le1, le2, gsem0, gsem1, ssem0, ssem1):
  cid = lax.axis_index("c")
  sid = lax.axis_index("s")
  wid = sid * NC + cid
  rows_bufs = (rows0, rows1)
  gsems = (gsem0, gsem1)
  ssems = (ssem0, ssem1)
  cbs = (cb0, cb1, cb2)
  ebs = (eb0, eb1, eb2)
  lcs = (lc0, lc1, lc2)
  les = (le0, le1, le2)
  ebase = wid * CHUNKS

  def _zrow(e, _):
    for k in range(8):
      rows0[e, pl.ds(k * 16, 16)] = jnp.zeros((16,), jnp.float32)
    return 0
  lax.fori_loop(0, CH, _zrow, 0, unroll=4)
  for i in range(ROWS_PER_TILE // CH):
    pltpu.sync_copy(rows0,
                    acc_sh.at[pl.ds(sid * ROWS_PER_TILE + i * CH, CH)])
  rem = ROWS_PER_TILE % CH
  if rem:
    pltpu.sync_copy(
        rows0.at[pl.ds(0, rem)],
        acc_sh.at[pl.ds(sid * ROWS_PER_TILE + (ROWS_PER_TILE // CH) * CH,
                        rem)])

  plsc.subcore_barrier()

  def _load(j, p):
    pltpu.async_copy(comb_hbm.at[j], cbs[p], lcs[p])
    pltpu.async_copy(ew_hbm.at[j], ebs[p], les[p])

  def _load_wait(p):
    pltpu.make_async_copy(comb_hbm.at[ebase], cbs[p], lcs[p]).wait()
    pltpu.make_async_copy(ew_hbm.at[ebase], ebs[p], les[p]).wait()

  _load(ebase + 0, 0)
  _load(ebase + 1, 1)
  _load_wait(0)
  pltpu.async_copy(h_hbm.at[cb0.at[0]], rows0, gsem0)

  def _six(i, _):
    for u in range(6):
      j = i * 6 + u
      b = u % 2
      p = u % 3
      pn = (u + 1) % 3
      pm = (u + 2) % 3
      rows = rows_bufs[b]
      cb = cbs[p]
      eb = ebs[p]
      pltpu.make_async_copy(h_hbm.at[cb.at[0]], rows, gsems[b]).wait()

      @pl.when(j >= 1)
      def _():
        pltpu.make_async_copy(rows_bufs[1 - b], acc_sh.at[cbs[pm].at[1]],
                              ssems[1 - b]).wait()

      @pl.when(j + 1 < CHUNKS)
      def _():
        _load_wait(pn)
        pltpu.async_copy(h_hbm.at[cbs[pn].at[0]], rows_bufs[1 - b],
                         gsems[1 - b])

      def _scale(g, _):
        wv = eb[0, pl.ds(g * 16, 16)]
        for t in range(16):
          w = wv[t]
          e = g * 16 + t
          for k in range(8):
            rows[e, pl.ds(k * 16, 16)] = rows[e, pl.ds(k * 16, 16)] * w
        return 0
      lax.fori_loop(0, CH // 16, _scale, 0)

      pltpu.async_copy(rows, acc_sh.at[cb.at[1]], ssems[b], add=True)

      @pl.when(j + 2 < CHUNKS)
      def _():
        _load(ebase + j + 2, pm)
    return 0

  lax.fori_loop(0, CHUNKS // 6, _six, 0)
  pltpu.make_async_copy(rows_bufs[(CHUNKS - 1) % 2],
                        acc_sh.at[cbs[(CHUNKS - 1) % 3].at[1]],
                        ssems[(CHUNKS - 1) % 2]).wait()
  plsc.subcore_barrier()

  @pl.when(cid == 0)
  def _():
    pltpu.sync_copy(acc_sh.at[pl.ds(sid * ROWS_PER_TILE, ROWS_PER_TILE)],
                    out0_hbm.at[pl.ds(sid * ROWS_PER_TILE, ROWS_PER_TILE)])

  @pl.when(cid == 1)
  def _():
    pltpu.sync_copy(acc_sh.at[pl.ds(sid * ROWS_PER_TILE, ROWS_PER_TILE)],
                    out1_hbm.at[pl.ds(sid * ROWS_PER_TILE, ROWS_PER_TILE)])


_edge_aggregate = functools.partial(
    pl.kernel,
    out_type=(jax.ShapeDtypeStruct((N_PAD, D), jnp.float32),
              jax.ShapeDtypeStruct((N_PAD, D), jnp.float32)),
    mesh=plsc.VectorSubcoreMesh(core_axis_name="c", subcore_axis_name="s"),
    scratch_types=[
        pltpu.VMEM((2, CH), jnp.int32),
        pltpu.VMEM((2, CH), jnp.int32),
        pltpu.VMEM((2, CH), jnp.int32),
        pltpu.VMEM((1, CH), jnp.float32),
        pltpu.VMEM((1, CH), jnp.float32),
        pltpu.VMEM((1, CH), jnp.float32),
        pltpu.VMEM((CH, D), jnp.float32),
        pltpu.VMEM((CH, D), jnp.float32),
        pltpu.VMEM_SHARED((N_PAD, D), jnp.float32),
        pltpu.SemaphoreType.DMA,
        pltpu.SemaphoreType.DMA,
        pltpu.SemaphoreType.DMA,
        pltpu.SemaphoreType.DMA,
        pltpu.SemaphoreType.DMA,
        pltpu.SemaphoreType.DMA,
        pltpu.SemaphoreType.DMA,
        pltpu.SemaphoreType.DMA,
        pltpu.SemaphoreType.DMA,
        pltpu.SemaphoreType.DMA,
    ],
)(_edge_body)


_MB = 1024
_GRID = N_PAD // _MB


def _proj_body(x_ref, w_ref, b_ref, o_ref):
  o_ref[...] = _dot_t(x_ref[...], w_ref[...]) + b_ref[...]


def _proj(x, w, b2):
  return pl.pallas_call(
      _proj_body,
      grid=(_GRID,),
      in_specs=[
          pl.BlockSpec((_MB, D), lambda i: (i, 0)),
          pl.BlockSpec((D, D), lambda i: (0, 0)),
          pl.BlockSpec((1, D), lambda i: (0, 0)),
      ],
      out_specs=pl.BlockSpec((_MB, D), lambda i: (i, 0)),
      out_shape=jax.ShapeDtypeStruct((N_PAD, D), jnp.float32),
  )(x, w, b2)


def _combine_body(p0_ref, p1_ref, h_ref, wrel_ref, brel_ref, wroot_ref, o_ref):
  aggr = p0_ref[...] + p1_ref[...]
  t = _dot_t(aggr, wrel_ref[...]) + _dot_t(h_ref[...], wroot_ref[...]) \
      + brel_ref[...]
  o_ref[...] = jnp.maximum(t, 0.0)


def _combine(p0, p1, h, wrel, brel2, wroot):
  return pl.pallas_call(
      _combine_body,
      grid=(_GRID,),
      in_specs=[
          pl.BlockSpec((_MB, D), lambda i: (i, 0)),
          pl.BlockSpec((_MB, D), lambda i: (i, 0)),
          pl.BlockSpec((_MB, D), lambda i: (i, 0)),
          pl.BlockSpec((D, D), lambda i: (0, 0)),
          pl.BlockSpec((1, D), lambda i: (0, 0)),
          pl.BlockSpec((D, D), lambda i: (0, 0)),
      ],
      out_specs=pl.BlockSpec((_MB, D), lambda i: (i, 0)),
      out_shape=jax.ShapeDtypeStruct((N_PAD, D), jnp.float32),
  )(p0, p1, h, wrel, brel2, wroot)


def _final_body(h_ref, b2_ref, wcls_ref, bcls_ref, logits_ref, g_ref,
                sums_acc, cnt_acc):
  i = pl.program_id(0)

  @pl.when(i == 0)
  def _():
    sums_acc[...] = jnp.zeros_like(sums_acc)
    cnt_acc[...] = jnp.zeros_like(cnt_acc)

  onehot = (b2_ref[...] == lax.broadcasted_iota(jnp.int32, (1, N_GRAPHS), 1)
            ).astype(jnp.float32)
  sums_acc[...] += lax.dot_general(
      onehot, h_ref[...], (((0,), (0,)), ((), ())),
      preferred_element_type=jnp.float32, precision=lax.Precision.HIGHEST)
  cnt_acc[...] += lax.dot_general(
      onehot, jnp.ones((_MB, 1), jnp.float32), (((0,), (0,)), ((), ())),
      preferred_element_type=jnp.float32, precision=lax.Precision.HIGHEST)

  @pl.when(i == _GRID - 1)
  def _():
    g = sums_acc[...] / jnp.maximum(cnt_acc[...], 1.0)
    g_ref[...] = g
    logits_ref[...] = _dot_t(g, wcls_ref[...]) + bcls_ref[...]


def _final(h, batch2d, wcls, bcls2):
  return pl.pallas_call(
      _final_body,
      grid=(_GRID,),
      in_specs=[
          pl.BlockSpec((_MB, D), lambda i: (i, 0)),
          pl.BlockSpec((_MB, 1), lambda i: (i, 0)),
          pl.BlockSpec((N_CLASSES, D), lambda i: (0, 0)),
          pl.BlockSpec((1, N_CLASSES), lambda i: (0, 0)),
      ],
      out_specs=[
          pl.BlockSpec((N_GRAPHS, N_CLASSES), lambda i: (0, 0)),
          pl.BlockSpec((N_GRAPHS, D), lambda i: (0, 0)),
      ],
      out_shape=[
          jax.ShapeDtypeStruct((N_GRAPHS, N_CLASSES), jnp.float32),
          jax.ShapeDtypeStruct((N_GRAPHS, D), jnp.float32),
      ],
      scratch_shapes=[
          pltpu.VMEM((N_GRAPHS, D), jnp.float32),
          pltpu.VMEM((N_GRAPHS, 1), jnp.float32),
      ],
  )(h, batch2d, wcls, bcls2)


def kernel(x_nodes, edge_index, edge_weight, batch, W_proj, b_proj,
           Wrel0, brel0, Wroot0, Wrel1, brel1, Wroot1, Wcls, bcls):
  pad = E_PAD - edge_weight.shape[0]
  src = jnp.concatenate([edge_index[0], jnp.zeros((pad,), jnp.int32)])
  dst = jnp.concatenate([edge_index[1], jnp.zeros((pad,), jnp.int32)])
  ew = jnp.concatenate([edge_weight, jnp.zeros((pad,), jnp.float32)])
  comb = jnp.stack([src.reshape(NW * CHUNKS, CH),
                    dst.reshape(NW * CHUNKS, CH)], axis=1)
  ew3 = ew.reshape(NW * CHUNKS, 1, CH)
  x_nodes = jnp.pad(x_nodes, ((0, N_PAD - N), (0, 0)))
  batch2d = jnp.pad(batch, (0, N_PAD - N), constant_values=N_GRAPHS)[:, None]

  h0 = _proj(x_nodes, W_proj, b_proj[None, :])
  p0a, p0b = _edge_aggregate(comb, ew3, h0)
  h1 = _combine(p0a, p0b, h0, Wrel0, brel0[None, :], Wroot0)
  p1a, p1b = _edge_aggregate(comb, ew3, h1)
  h2 = _combine(p1a, p1b, h1, Wrel1, brel1[None, :], Wroot1)
  logits, g = _final(h2, batch2d, Wcls, bcls[None, :])
  return logits, g


# speedup vs baseline: 7.0451x; 1.0611x over previous
import functools

import jax
import jax.numpy as jnp
from jax import lax
from jax.experimental import pallas as pl
from jax.experimental.pallas import tpu as pltpu
from jax.experimental.pallas import tpu_sc as plsc

N = 10000
N_PAD = 10240
D = 128
N_GRAPHS = 64
N_CLASSES = 16

NC = 2
NS = 16
NW = NC * NS
CH = 112
CHUNKS = 90
GRP = 3
E_PAD = NW * CHUNKS * CH
ROWS_PER_TILE = N_PAD // NS


def _dot_t(a, b):
  return lax.dot_general(a, b, (((1,), (1,)), ((), ())),
                         preferred_element_type=jnp.float32,
                         precision=lax.Precision.HIGHEST)


def _edge_body(comb_hbm, ew_hbm, h_hbm, out0_hbm, out1_hbm,
               cb0, cb1, cb2, cb3, cb4, cb5,
               eb0, eb1, eb2, eb3, eb4, eb5,
               rows0, rows1, rows2, acc_sh,
               lc0, lc1, lc2, lc3, lc4, lc5,
               le0, le1, le2, le3, le4, le5,
               gsem0, gsem1, gsem2, ssem0, ssem1, ssem2):
  cid = lax.axis_index("c")
  sid = lax.axis_index("s")
  wid = sid * NC + cid
  rows_bufs = (rows0, rows1, rows2)
  gsems = (gsem0, gsem1, gsem2)
  ssems = (ssem0, ssem1, ssem2)
  cbs = (cb0, cb1, cb2, cb3, cb4, cb5)
  ebs = (eb0, eb1, eb2, eb3, eb4, eb5)
  lcs = (lc0, lc1, lc2, lc3, lc4, lc5)
  les = (le0, le1, le2, le3, le4, le5)
  ebase = wid * CHUNKS

  def _zrow(e, _):
    for k in range(8):
      rows0[e, pl.ds(k * 16, 16)] = jnp.zeros((16,), jnp.float32)
    return 0
  lax.fori_loop(0, CH, _zrow, 0, unroll=4)
  for i in range(ROWS_PER_TILE // CH):
    pltpu.sync_copy(rows0,
                    acc_sh.at[pl.ds(sid * ROWS_PER_TILE + i * CH, CH)])
  rem = ROWS_PER_TILE % CH
  if rem:
    pltpu.sync_copy(
        rows0.at[pl.ds(0, rem)],
        acc_sh.at[pl.ds(sid * ROWS_PER_TILE + (ROWS_PER_TILE // CH) * CH,
                        rem)])

  plsc.subcore_barrier()

  def _load(j, p):
    pltpu.async_copy(comb_hbm.at[j], cbs[p], lcs[p])
    pltpu.async_copy(ew_hbm.at[j], ebs[p], les[p])

  def _load_wait(p):
    pltpu.make_async_copy(comb_hbm.at[ebase], cbs[p], lcs[p]).wait()
    pltpu.make_async_copy(ew_hbm.at[ebase], ebs[p], les[p]).wait()

  for k in range(4):
    _load(ebase + k, k)
  _load_wait(0)
  pltpu.async_copy(h_hbm.at[cb0.at[0]], rows0, gsem0)
  _load_wait(1)
  pltpu.async_copy(h_hbm.at[cb1.at[0]], rows1, gsem1)

  def _six(i, _):
    for u in range(6):
      j = i * 6 + u
      b = u % 3
      p = u % 6
      p1 = (u + 5) % 6
      p2 = (u + 2) % 6
      p4 = (u + 4) % 6
      rows = rows_bufs[b]
      pltpu.make_async_copy(h_hbm.at[cbs[p].at[0]], rows, gsems[b]).wait()

      @pl.when(j >= 1)
      def _():
        pltpu.make_async_copy(rows_bufs[(b + 2) % 3],
                              acc_sh.at[cbs[p1].at[1]],
                              ssems[(b + 2) % 3]).wait()

      @pl.when(j + 2 < CHUNKS)
      def _():
        _load_wait(p2)
        pltpu.async_copy(h_hbm.at[cbs[p2].at[0]], rows_bufs[(b + 2) % 3],
                         gsems[(b + 2) % 3])

      def _scale(g, _):
        wv = ebs[p][0, pl.ds(g * 16, 16)]
        for t in range(16):
          w = wv[t]
          e = g * 16 + t
          for k in range(8):
            rows[e, pl.ds(k * 16, 16)] = rows[e, pl.ds(k * 16, 16)] * w
        return 0
      lax.fori_loop(0, CH // 16, _scale, 0)

      pltpu.async_copy(rows, acc_sh.at[cbs[p].at[1]], ssems[b], add=True)

      @pl.when(j + 4 < CHUNKS)
      def _():
        _load(ebase + j + 4, p4)
    return 0

  lax.fori_loop(0, CHUNKS // 6, _six, 0)
  pltpu.make_async_copy(rows_bufs[(CHUNKS - 1) % 3],
                        acc_sh.at[cbs[(CHUNKS - 1) % 6].at[1]],
                        ssems[(CHUNKS - 1) % 3]).wait()
  plsc.subcore_barrier()

  @pl.when(cid == 0)
  def _():
    pltpu.sync_copy(acc_sh.at[pl.ds(sid * ROWS_PER_TILE, ROWS_PER_TILE)],
                    out0_hbm.at[pl.ds(sid * ROWS_PER_TILE, ROWS_PER_TILE)])

  @pl.when(cid == 1)
  def _():
    pltpu.sync_copy(acc_sh.at[pl.ds(sid * ROWS_PER_TILE, ROWS_PER_TILE)],
                    out1_hbm.at[pl.ds(sid * ROWS_PER_TILE, ROWS_PER_TILE)])


_edge_aggregate = functools.partial(
    pl.kernel,
    out_type=(jax.ShapeDtypeStruct((N_PAD, D), jnp.float32),
              jax.ShapeDtypeStruct((N_PAD, D), jnp.float32)),
    mesh=plsc.VectorSubcoreMesh(core_axis_name="c", subcore_axis_name="s"),
    scratch_types=(
        [pltpu.VMEM((2, CH), jnp.int32) for _ in range(6)]
        + [pltpu.VMEM((1, CH), jnp.float32) for _ in range(6)]
        + [pltpu.VMEM((CH, D), jnp.float32) for _ in range(3)]
        + [pltpu.VMEM_SHARED((N_PAD, D), jnp.float32)]
        + [pltpu.SemaphoreType.DMA for _ in range(18)]
    ),
)(_edge_body)


_MB = 1024
_GRID = N_PAD // _MB


def _proj_body(x_ref, w_ref, b_ref, o_ref):
  o_ref[...] = _dot_t(x_ref[...], w_ref[...]) + b_ref[...]


def _proj(x, w, b2):
  return pl.pallas_call(
      _proj_body,
      grid=(_GRID,),
      in_specs=[
          pl.BlockSpec((_MB, D), lambda i: (i, 0)),
          pl.BlockSpec((D, D), lambda i: (0, 0)),
          pl.BlockSpec((1, D), lambda i: (0, 0)),
      ],
      out_specs=pl.BlockSpec((_MB, D), lambda i: (i, 0)),
      out_shape=jax.ShapeDtypeStruct((N_PAD, D), jnp.float32),
  )(x, w, b2)


def _combine_body(p0_ref, p1_ref, h_ref, wrel_ref, brel_ref, wroot_ref, o_ref):
  aggr = p0_ref[...] + p1_ref[...]
  t = _dot_t(aggr, wrel_ref[...]) + _dot_t(h_ref[...], wroot_ref[...]) \
      + brel_ref[...]
  o_ref[...] = jnp.maximum(t, 0.0)


def _combine(p0, p1, h, wrel, brel2, wroot):
  return pl.pallas_call(
      _combine_body,
      grid=(_GRID,),
      in_specs=[
          pl.BlockSpec((_MB, D), lambda i: (i, 0)),
          pl.BlockSpec((_MB, D), lambda i: (i, 0)),
          pl.BlockSpec((_MB, D), lambda i: (i, 0)),
          pl.BlockSpec((D, D), lambda i: (0, 0)),
          pl.BlockSpec((1, D), lambda i: (0, 0)),
          pl.BlockSpec((D, D), lambda i: (0, 0)),
      ],
      out_specs=pl.BlockSpec((_MB, D), lambda i: (i, 0)),
      out_shape=jax.ShapeDtypeStruct((N_PAD, D), jnp.float32),
  )(p0, p1, h, wrel, brel2, wroot)


def _final_body(h_ref, b2_ref, wcls_ref, bcls_ref, logits_ref, g_ref,
                sums_acc, cnt_acc):
  i = pl.program_id(0)

  @pl.when(i == 0)
  def _():
    sums_acc[...] = jnp.zeros_like(sums_acc)
    cnt_acc[...] = jnp.zeros_like(cnt_acc)

  onehot = (b2_ref[...] == lax.broadcasted_iota(jnp.int32, (1, N_GRAPHS), 1)
            ).astype(jnp.float32)
  sums_acc[...] += lax.dot_general(
      onehot, h_ref[...], (((0,), (0,)), ((), ())),
      preferred_element_type=jnp.float32, precision=lax.Precision.HIGHEST)
  cnt_acc[...] += lax.dot_general(
      onehot, jnp.ones((_MB, 1), jnp.float32), (((0,), (0,)), ((), ())),
      preferred_element_type=jnp.float32, precision=lax.Precision.HIGHEST)

  @pl.when(i == _GRID - 1)
  def _():
    g = sums_acc[...] / jnp.maximum(cnt_acc[...], 1.0)
    g_ref[...] = g
    logits_ref[...] = _dot_t(g, wcls_ref[...]) + bcls_ref[...]


def _final(h, batch2d, wcls, bcls2):
  return pl.pallas_call(
      _final_body,
      grid=(_GRID,),
      in_specs=[
          pl.BlockSpec((_MB, D), lambda i: (i, 0)),
          pl.BlockSpec((_MB, 1), lambda i: (i, 0)),
          pl.BlockSpec((N_CLASSES, D), lambda i: (0, 0)),
          pl.BlockSpec((1, N_CLASSES), lambda i: (0, 0)),
      ],
      out_specs=[
          pl.BlockSpec((N_GRAPHS, N_CLASSES), lambda i: (0, 0)),
          pl.BlockSpec((N_GRAPHS, D), lambda i: (0, 0)),
      ],
      out_shape=[
          jax.ShapeDtypeStruct((N_GRAPHS, N_CLASSES), jnp.float32),
          jax.ShapeDtypeStruct((N_GRAPHS, D), jnp.float32),
      ],
      scratch_shapes=[
          pltpu.VMEM((N_GRAPHS, D), jnp.float32),
          pltpu.VMEM((N_GRAPHS, 1), jnp.float32),
      ],
  )(h, batch2d, wcls, bcls2)


def kernel(x_nodes, edge_index, edge_weight, batch, W_proj, b_proj,
           Wrel0, brel0, Wroot0, Wrel1, brel1, Wroot1, Wcls, bcls):
  pad = E_PAD - edge_weight.shape[0]
  src = jnp.concatenate([edge_index[0], jnp.zeros((pad,), jnp.int32)])
  dst = jnp.concatenate([edge_index[1], jnp.zeros((pad,), jnp.int32)])
  ew = jnp.concatenate([edge_weight, jnp.zeros((pad,), jnp.float32)])
  comb = jnp.stack([src.reshape(NW * CHUNKS, CH),
                    dst.reshape(NW * CHUNKS, CH)], axis=1)
  ew3 = ew.reshape(NW * CHUNKS, 1, CH)
  x_nodes = jnp.pad(x_nodes, ((0, N_PAD - N), (0, 0)))
  batch2d = jnp.pad(batch, (0, N_PAD - N), constant_values=N_GRAPHS)[:, None]

  h0 = _proj(x_nodes, W_proj, b_proj[None, :])
  p0a, p0b = _edge_aggregate(comb, ew3, h0)
  h1 = _combine(p0a, p0b, h0, Wrel0, brel0[None, :], Wroot0)
  p1a, p1b = _edge_aggregate(comb, ew3, h1)
  h2 = _combine(p1a, p1b, h1, Wrel1, brel1[None, :], Wroot1)
  logits, g = _final(h2, batch2d, Wcls, bcls[None, :])
  return logits, g


# speedup vs baseline: 7.1499x; 1.0149x over previous
import functools

import jax
import jax.numpy as jnp
from jax import lax
from jax.experimental import pallas as pl
from jax.experimental.pallas import tpu as pltpu
from jax.experimental.pallas import tpu_sc as plsc

N = 10000
N_PAD = 10240
D = 128
N_GRAPHS = 64
N_CLASSES = 16

NC = 2
NS = 16
NW = NC * NS
CH = 112
CHUNKS = 90
GRP = 3
E_PAD = NW * CHUNKS * CH
ROWS_PER_TILE = N_PAD // NS


def _dot_t(a, b):
  return lax.dot_general(a, b, (((1,), (1,)), ((), ())),
                         preferred_element_type=jnp.float32,
                         precision=lax.Precision.HIGHEST)


def _edge_body(comb_hbm, ew_hbm, h_hbm, out0_hbm, out1_hbm,
               cb0, cb1, cb2, cb3, cb4, cb5,
               eb0, eb1, eb2, eb3, eb4, eb5,
               rows0, rows1, rows2, acc_sh,
               lc0, lc1, lc2, lc3, lc4, lc5,
               le0, le1, le2, le3, le4, le5,
               gsem0, gsem1, gsem2, ssem0, ssem1, ssem2):
  cid = lax.axis_index("c")
  sid = lax.axis_index("s")
  wid = sid * NC + cid
  rows_bufs = (rows0, rows1, rows2)
  gsems = (gsem0, gsem1, gsem2)
  ssems = (ssem0, ssem1, ssem2)
  cbs = (cb0, cb1, cb2, cb3, cb4, cb5)
  ebs = (eb0, eb1, eb2, eb3, eb4, eb5)
  lcs = (lc0, lc1, lc2, lc3, lc4, lc5)
  les = (le0, le1, le2, le3, le4, le5)
  ebase = wid * CHUNKS

  def _zrow(e, _):
    for k in range(8):
      rows0[e, pl.ds(k * 16, 16)] = jnp.zeros((16,), jnp.float32)
    return 0
  lax.fori_loop(0, CH, _zrow, 0, unroll=4)
  for i in range(ROWS_PER_TILE // CH):
    pltpu.sync_copy(rows0,
                    acc_sh.at[pl.ds(sid * ROWS_PER_TILE + i * CH, CH)])
  rem = ROWS_PER_TILE % CH
  if rem:
    pltpu.sync_copy(
        rows0.at[pl.ds(0, rem)],
        acc_sh.at[pl.ds(sid * ROWS_PER_TILE + (ROWS_PER_TILE // CH) * CH,
                        rem)])

  plsc.subcore_barrier()

  def _load(j, p):
    pltpu.async_copy(comb_hbm.at[j], cbs[p], lcs[p])
    pltpu.async_copy(ew_hbm.at[j], ebs[p], les[p])

  def _load_wait(p):
    pltpu.make_async_copy(comb_hbm.at[ebase], cbs[p], lcs[p]).wait()
    pltpu.make_async_copy(ew_hbm.at[ebase], ebs[p], les[p]).wait()

  for k in range(4):
    _load(ebase + k, k)
  _load_wait(0)
  pltpu.async_copy(h_hbm.at[cb0.at[0]], rows0, gsem0)
  _load_wait(1)
  pltpu.async_copy(h_hbm.at[cb1.at[0]], rows1, gsem1)

  def _six(i, _):
    for u in range(6):
      j = i * 6 + u
      b = u % 3
      p = u % 6
      p1 = (u + 5) % 6
      p2 = (u + 2) % 6
      p4 = (u + 4) % 6
      rows = rows_bufs[b]
      pltpu.make_async_copy(h_hbm.at[cbs[p].at[0]], rows, gsems[b]).wait()

      def _scale(g, _):
        wv = ebs[p][0, pl.ds(g * 16, 16)]
        for t in range(16):
          w = wv[t]
          e = g * 16 + t
          for k in range(8):
            rows[e, pl.ds(k * 16, 16)] = rows[e, pl.ds(k * 16, 16)] * w
        return 0
      lax.fori_loop(0, CH // 16, _scale, 0)

      @pl.when(j >= 1)
      def _():
        pltpu.make_async_copy(rows_bufs[(b + 2) % 3],
                              acc_sh.at[cbs[p1].at[1]],
                              ssems[(b + 2) % 3]).wait()

      @pl.when(j + 2 < CHUNKS)
      def _():
        _load_wait(p2)
        pltpu.async_copy(h_hbm.at[cbs[p2].at[0]], rows_bufs[(b + 2) % 3],
                         gsems[(b + 2) % 3])

      pltpu.async_copy(rows, acc_sh.at[cbs[p].at[1]], ssems[b], add=True)

      @pl.when(j + 4 < CHUNKS)
      def _():
        _load(ebase + j + 4, p4)
    return 0

  lax.fori_loop(0, CHUNKS // 6, _six, 0)
  pltpu.make_async_copy(rows_bufs[(CHUNKS - 1) % 3],
                        acc_sh.at[cbs[(CHUNKS - 1) % 6].at[1]],
                        ssems[(CHUNKS - 1) % 3]).wait()
  plsc.subcore_barrier()

  @pl.when(cid == 0)
  def _():
    pltpu.sync_copy(acc_sh.at[pl.ds(sid * ROWS_PER_TILE, ROWS_PER_TILE)],
                    out0_hbm.at[pl.ds(sid * ROWS_PER_TILE, ROWS_PER_TILE)])

  @pl.when(cid == 1)
  def _():
    pltpu.sync_copy(acc_sh.at[pl.ds(sid * ROWS_PER_TILE, ROWS_PER_TILE)],
                    out1_hbm.at[pl.ds(sid * ROWS_PER_TILE, ROWS_PER_TILE)])


_edge_aggregate = functools.partial(
    pl.kernel,
    out_type=(jax.ShapeDtypeStruct((N_PAD, D), jnp.float32),
              jax.ShapeDtypeStruct((N_PAD, D), jnp.float32)),
    mesh=plsc.VectorSubcoreMesh(core_axis_name="c", subcore_axis_name="s"),
    scratch_types=(
        [pltpu.VMEM((2, CH), jnp.int32) for _ in range(6)]
        + [pltpu.VMEM((1, CH), jnp.float32) for _ in range(6)]
        + [pltpu.VMEM((CH, D), jnp.float32) for _ in range(3)]
        + [pltpu.VMEM_SHARED((N_PAD, D), jnp.float32)]
        + [pltpu.SemaphoreType.DMA for _ in range(18)]
    ),
)(_edge_body)


_MB = 1024
_GRID = N_PAD // _MB


def _proj_body(x_ref, w_ref, b_ref, o_ref):
  o_ref[...] = _dot_t(x_ref[...], w_ref[...]) + b_ref[...]


def _proj(x, w, b2):
  return pl.pallas_call(
      _proj_body,
      grid=(_GRID,),
      in_specs=[
          pl.BlockSpec((_MB, D), lambda i: (i, 0)),
          pl.BlockSpec((D, D), lambda i: (0, 0)),
          pl.BlockSpec((1, D), lambda i: (0, 0)),
      ],
      out_specs=pl.BlockSpec((_MB, D), lambda i: (i, 0)),
      out_shape=jax.ShapeDtypeStruct((N_PAD, D), jnp.float32),
  )(x, w, b2)


def _combine_body(p0_ref, p1_ref, h_ref, wrel_ref, brel_ref, wroot_ref, o_ref):
  aggr = p0_ref[...] + p1_ref[...]
  t = _dot_t(aggr, wrel_ref[...]) + _dot_t(h_ref[...], wroot_ref[...]) \
      + brel_ref[...]
  o_ref[...] = jnp.maximum(t, 0.0)


def _combine(p0, p1, h, wrel, brel2, wroot):
  return pl.pallas_call(
      _combine_body,
      grid=(_GRID,),
      in_specs=[
          pl.BlockSpec((_MB, D), lambda i: (i, 0)),
          pl.BlockSpec((_MB, D), lambda i: (i, 0)),
          pl.BlockSpec((_MB, D), lambda i: (i, 0)),
          pl.BlockSpec((D, D), lambda i: (0, 0)),
          pl.BlockSpec((1, D), lambda i: (0, 0)),
          pl.BlockSpec((D, D), lambda i: (0, 0)),
      ],
      out_specs=pl.BlockSpec((_MB, D), lambda i: (i, 0)),
      out_shape=jax.ShapeDtypeStruct((N_PAD, D), jnp.float32),
  )(p0, p1, h, wrel, brel2, wroot)


def _final_body(h_ref, b2_ref, wcls_ref, bcls_ref, logits_ref, g_ref,
                sums_acc, cnt_acc):
  i = pl.program_id(0)

  @pl.when(i == 0)
  def _():
    sums_acc[...] = jnp.zeros_like(sums_acc)
    cnt_acc[...] = jnp.zeros_like(cnt_acc)

  onehot = (b2_ref[...] == lax.broadcasted_iota(jnp.int32, (1, N_GRAPHS), 1)
            ).astype(jnp.float32)
  sums_acc[...] += lax.dot_general(
      onehot, h_ref[...], (((0,), (0,)), ((), ())),
      preferred_element_type=jnp.float32, precision=lax.Precision.HIGHEST)
  cnt_acc[...] += lax.dot_general(
      onehot, jnp.ones((_MB, 1), jnp.float32), (((0,), (0,)), ((), ())),
      preferred_element_type=jnp.float32, precision=lax.Precision.HIGHEST)

  @pl.when(i == _GRID - 1)
  def _():
    g = sums_acc[...] / jnp.maximum(cnt_acc[...], 1.0)
    g_ref[...] = g
    logits_ref[...] = _dot_t(g, wcls_ref[...]) + bcls_ref[...]


def _final(h, batch2d, wcls, bcls2):
  return pl.pallas_call(
      _final_body,
      grid=(_GRID,),
      in_specs=[
          pl.BlockSpec((_MB, D), lambda i: (i, 0)),
          pl.BlockSpec((_MB, 1), lambda i: (i, 0)),
          pl.BlockSpec((N_CLASSES, D), lambda i: (0, 0)),
          pl.BlockSpec((1, N_CLASSES), lambda i: (0, 0)),
      ],
      out_specs=[
          pl.BlockSpec((N_GRAPHS, N_CLASSES), lambda i: (0, 0)),
          pl.BlockSpec((N_GRAPHS, D), lambda i: (0, 0)),
      ],
      out_shape=[
          jax.ShapeDtypeStruct((N_GRAPHS, N_CLASSES), jnp.float32),
          jax.ShapeDtypeStruct((N_GRAPHS, D), jnp.float32),
      ],
      scratch_shapes=[
          pltpu.VMEM((N_GRAPHS, D), jnp.float32),
          pltpu.VMEM((N_GRAPHS, 1), jnp.float32),
      ],
  )(h, batch2d, wcls, bcls2)


def kernel(x_nodes, edge_index, edge_weight, batch, W_proj, b_proj,
           Wrel0, brel0, Wroot0, Wrel1, brel1, Wroot1, Wcls, bcls):
  pad = E_PAD - edge_weight.shape[0]
  src = jnp.concatenate([edge_index[0], jnp.zeros((pad,), jnp.int32)])
  dst = jnp.concatenate([edge_index[1], jnp.zeros((pad,), jnp.int32)])
  ew = jnp.concatenate([edge_weight, jnp.zeros((pad,), jnp.float32)])
  comb = jnp.stack([src.reshape(NW * CHUNKS, CH),
                    dst.reshape(NW * CHUNKS, CH)], axis=1)
  ew3 = ew.reshape(NW * CHUNKS, 1, CH)
  x_nodes = jnp.pad(x_nodes, ((0, N_PAD - N), (0, 0)))
  batch2d = jnp.pad(batch, (0, N_PAD - N), constant_values=N_GRAPHS)[:, None]

  h0 = _proj(x_nodes, W_proj, b_proj[None, :])
  p0a, p0b = _edge_aggregate(comb, ew3, h0)
  h1 = _combine(p0a, p0b, h0, Wrel0, brel0[None, :], Wroot0)
  p1a, p1b = _edge_aggregate(comb, ew3, h1)
  h2 = _combine(p1a, p1b, h1, Wrel1, brel1[None, :], Wroot1)
  logits, g = _final(h2, batch2d, Wcls, bcls[None, :])
  return logits, g
